# baseline (speedup 1.0000x reference)
import functools

import jax
import jax.numpy as jnp
from jax import lax
from jax.experimental import pallas as pl
from jax.experimental.pallas import tpu as pltpu
from jax.experimental.pallas import tpu_sc as plsc

_N_BINS = 10
_BB = 32
_NC = 2
_NS = 16
_NW = _NC * _NS
_L = 16


def _tc_bins_kernel(pred_ref, truth_ref, bins_ref):
    x = jnp.swapaxes(pred_ref[...], 1, 2)
    tv = truth_ref[...]
    t = tv[:, None, :]

    mask = x < t
    r = jnp.sum(mask.astype(jnp.float32), axis=1).astype(jnp.int32)
    a = jnp.max(jnp.where(mask, x, -jnp.inf), axis=1)
    b = jnp.min(jnp.where(mask, jnp.inf, x), axis=1)

    base = jnp.where(r >= 1, 1 + jnp.minimum((r - 1) // 10, 9), 0)
    base = base + jnp.where(r == 100, 1, 0)

    amb = (r % 10 == 0) & (r >= 10) & (r <= 90)
    kf = (r // 10).astype(jnp.float32)
    qv = kf * jnp.float32(0.1)
    idx = qv * jnp.float32(99.0)
    hw = idx - jnp.floor(idx)
    lw = jnp.float32(1.0) - hw
    interp = a * lw + b * hw
    m = base + jnp.where(amb & (interp < tv), 1, 0)

    bins_ref[...] = jnp.clip(m, 1, _N_BINS) - 1


def _tc_bins(prediction, truth):
    nb = prediction.shape[0]
    return pl.pallas_call(
        _tc_bins_kernel,
        grid=(nb // _BB,),
        in_specs=[
            pl.BlockSpec((_BB, 256, 100), lambda i: (i, 0, 0)),
            pl.BlockSpec((_BB, 256), lambda i: (i, 0)),
        ],
        out_specs=pl.BlockSpec((_BB, 256), lambda i: (i, 0)),
        out_shape=jax.ShapeDtypeStruct((nb, 256), jnp.int32),
    )(prediction, truth)


_sc_mesh = plsc.VectorSubcoreMesh(
    core_axis_name="c", subcore_axis_name="s", num_cores=_NC, num_subcores=_NS)


@functools.partial(
    pl.kernel,
    out_type=jax.ShapeDtypeStruct((_NW, _N_BINS, _L), jnp.float32),
    mesh=_sc_mesh,
    scratch_types=[
        pltpu.VMEM((16, 256), jnp.int32),
        pltpu.VMEM((_N_BINS, _L), jnp.float32),
    ],
)
def _sc_hist(bins_hbm, out_hbm, in_v, hist_v):
    wid = lax.axis_index("s") * _NC + lax.axis_index("c")
    pltpu.sync_copy(bins_hbm.at[pl.ds(wid * 16, 16)], in_v)

    one = jnp.ones((_L,), jnp.float32)
    zero = jnp.zeros((_L,), jnp.float32)

    def row_body(rr, accs):
        def col_body(cc, accs):
            v = in_v[rr, pl.ds(cc * _L, _L)]
            return tuple(
                acc + jnp.where(v == k, one, zero)
                for k, acc in enumerate(accs))
        return lax.fori_loop(0, 256 // _L, col_body, accs)

    accs = lax.fori_loop(
        0, 16, row_body,
        tuple(jnp.zeros((_L,), jnp.float32) for _ in range(_N_BINS)))

    for k, acc in enumerate(accs):
        hist_v[k, :] = acc
    pltpu.sync_copy(hist_v, out_hbm.at[wid])


@jax.jit
def kernel(prediction, truth):
    bins = _tc_bins(prediction, truth)
    part = _sc_hist(bins)
    return jnp.sum(part, axis=(0, 2))
